# baseline (speedup 1.0000x reference)
import functools

import jax
import jax.numpy as jnp
from jax import lax
from jax.experimental import pallas as pl
from jax.experimental.pallas import tpu as pltpu
from jax.experimental.pallas import tpu_sc as plsc

T = 8
G3 = 128 ** 3
N = T * G3
U = T * (G3 // 4)
DECAY = 0.95

NC, NS = 2, 16
NW = NC * NS
CHUNK = 32768
NCHUNK = N // CHUNK
CPT = NCHUNK // NW
BATCH = 2048
KPAD = BATCH + 64
SENTINEL = 0x7FFFFFFF
NBOUND = NCHUNK + 1
BPAD = 544

_MESH = plsc.VectorSubcoreMesh(core_axis_name="c", subcore_axis_name="s")


@functools.partial(
    pl.kernel,
    mesh=_MESH,
    compiler_params=pltpu.CompilerParams(needs_layout_passes=False),
    out_type=jax.ShapeDtypeStruct((N,), jnp.float32),
    scratch_types=[
        pltpu.VMEM((CHUNK,), jnp.float32),
        pltpu.VMEM((CHUNK,), jnp.float32),
        pltpu.VMEM((KPAD,), jnp.int32),
        pltpu.VMEM((KPAD,), jnp.int32),
        pltpu.VMEM((KPAD,), jnp.float32),
        pltpu.VMEM((KPAD,), jnp.float32),
        pltpu.VMEM((BPAD,), jnp.int32),
        pltpu.SemaphoreType.DMA,
        pltpu.SemaphoreType.DMA,
        pltpu.SemaphoreType.DMA,
        pltpu.SemaphoreType.DMA,
        pltpu.SemaphoreType.DMA,
        pltpu.SemaphoreType.DMA,
    ],
)
def _grid_update(mem_hbm, sk_hbm, sv_hbm, bnd_hbm, out_hbm,
                 buf0, buf1, kbuf0, kbuf1, vbuf0, vbuf1, bvec,
                 sin0, sin1, sout0, sout1, skv0, skv1):
    wid = lax.axis_index("s") * NC + lax.axis_index("c")
    pltpu.sync_copy(bnd_hbm, bvec)
    lanes = lax.iota(jnp.int32, 16)
    bufs = (buf0, buf1)
    kbufs = (kbuf0, kbuf1)
    vbufs = (vbuf0, vbuf1)
    sin = (sin0, sin1)
    sout = (sout0, sout1)
    skv = (skv0, skv1)

    def _bnd(pos):
        return plsc.load_gather(bvec, [jnp.zeros((16,), jnp.int32) + pos])[0]

    def chunk_cells(j):
        return (wid * CPT + j) * CHUNK

    def start_in(j, b):
        pltpu.async_copy(mem_hbm.at[pl.ds(chunk_cells(j), CHUNK)],
                         bufs[b], sin[b])

    def wait_in(j, b):
        pltpu.make_async_copy(mem_hbm.at[pl.ds(chunk_cells(j), CHUNK)],
                              bufs[b], sin[b]).wait()

    def start_out(j, b):
        pltpu.async_copy(bufs[b], out_hbm.at[pl.ds(chunk_cells(j), CHUNK)],
                         sout[b])

    def wait_out(j, b):
        pltpu.make_async_copy(bufs[b],
                              out_hbm.at[pl.ds(chunk_cells(j), CHUNK)],
                              sout[b]).wait()

    def start_kv(base, p):
        pltpu.async_copy(sk_hbm.at[pl.ds(base, KPAD)], kbufs[p], skv[p])
        pltpu.async_copy(sv_hbm.at[pl.ds(base, KPAD)], vbufs[p], skv[p])

    def wait_kv(base, p):
        pltpu.make_async_copy(sk_hbm.at[pl.ds(base, KPAD)],
                              kbufs[p], skv[p]).wait()
        pltpu.make_async_copy(sv_hbm.at[pl.ds(base, KPAD)],
                              vbufs[p], skv[p]).wait()

    def process_chunk(j, b):
        k = wid * CPT + j
        cb = k * CHUNK
        s = _bnd(k)
        e = _bnd(k + 1)
        astart = (s // 16) * 16
        nb = (e - astart + (BATCH - 1)) // BATCH
        buf = bufs[b]

        @pl.when(nb > 0)
        def _():
            start_kv(astart, 0)

            def win_pair(ibb, _):
                for p in range(2):
                    ib = ibb + p

                    @pl.when(ib < nb)
                    def _():
                        base = astart + ib * BATCH
                        wait_kv(base, p)

                        @pl.when(ib + 1 < nb)
                        def _():
                            start_kv(astart + (ib + 1) * BATCH, 1 - p)

                        kbuf = kbufs[p]
                        vbuf = vbufs[p]

                        def vec_body(i, _):
                            off = i * 16
                            ka = kbuf[pl.ds(off, 16)]
                            kb = plsc.load_gather(kbuf, [off + 1 + lanes])
                            v = vbuf[pl.ds(off, 16)]
                            pos = (base + off) + lanes
                            m = (ka != kb) & (pos >= s) & (pos < e)
                            lidx = jnp.minimum(jnp.maximum(ka - cb, 0),
                                               CHUNK - 1)
                            g = plsc.load_gather(buf, [lidx])
                            w = jnp.maximum(g * jnp.float32(DECAY), v)
                            plsc.store_scatter(buf, [lidx], w, mask=m)
                            return 0

                        lax.fori_loop(0, BATCH // 16, vec_body, 0)
                return 0

            lax.fori_loop(0, (nb + 1) // 2, lambda q, c: win_pair(q * 2, c), 0)

    start_in(0, 0)

    def ring_pair(jj, _):
        for b in range(2):
            j = jj + b
            wait_in(j, b)

            @pl.when((j + 1 < CPT) & (j >= 1))
            def _():
                wait_out(j - 1, 1 - b)

            @pl.when(j + 1 < CPT)
            def _():
                start_in(j + 1, 1 - b)

            process_chunk(j, b)
            start_out(j, b)
        return 0

    lax.fori_loop(0, CPT // 2, lambda q, c: ring_pair(q * 2, c), 0)
    wait_out(CPT - 2, 0)
    wait_out(CPT - 1, 1)


def kernel(mem, idx, val):
    keys = (idx.astype(jnp.int32)
            + (jnp.arange(T, dtype=jnp.int32) * G3)[:, None]).reshape(-1)
    sk, sv = lax.sort((keys, val.reshape(-1)), is_stable=False, num_keys=1)
    bnd = jnp.searchsorted(
        sk, jnp.arange(NBOUND, dtype=jnp.int32) * CHUNK, side="left"
    ).astype(jnp.int32)
    bnd = jnp.concatenate([bnd, jnp.zeros((BPAD - NBOUND,), jnp.int32)])
    skp = jnp.concatenate([sk, jnp.full((KPAD,), SENTINEL, jnp.int32)])
    svp = jnp.concatenate([sv, jnp.zeros((KPAD,), jnp.float32)])
    out = _grid_update(mem.reshape(-1), skp, svp, bnd)
    return out.reshape(T, G3)


# speedup vs baseline: 1.2782x; 1.2782x over previous
import functools

import jax
import jax.numpy as jnp
from jax import lax
from jax.experimental import pallas as pl
from jax.experimental.pallas import tpu as pltpu
from jax.experimental.pallas import tpu_sc as plsc

T = 8
G3 = 128 ** 3
N = T * G3
U = T * (G3 // 4)
DECAY = 0.95

NC, NS = 2, 16
NW = NC * NS
CHUNK = 32768
NCHUNK = N // CHUNK
CPT = NCHUNK // NW
CELLS = CHUNK // T
BATCH = 1024
KPAD = BATCH + 64
SENTINEL = 0x7FFFFFFF
NQ = NCHUNK + 1
BPAD = 4128

_MESH = plsc.VectorSubcoreMesh(core_axis_name="c", subcore_axis_name="s")


@functools.partial(
    pl.kernel,
    mesh=_MESH,
    compiler_params=pltpu.CompilerParams(needs_layout_passes=False),
    out_type=jax.ShapeDtypeStruct((N,), jnp.float32),
    scratch_types=[
        pltpu.VMEM((CHUNK,), jnp.float32),
        pltpu.VMEM((CHUNK,), jnp.float32),
        pltpu.VMEM((KPAD,), jnp.int32),
        pltpu.VMEM((KPAD,), jnp.int32),
        pltpu.VMEM((KPAD,), jnp.float32),
        pltpu.VMEM((KPAD,), jnp.float32),
        pltpu.VMEM((BPAD,), jnp.int32),
        pltpu.SemaphoreType.DMA,
        pltpu.SemaphoreType.DMA,
        pltpu.SemaphoreType.DMA,
        pltpu.SemaphoreType.DMA,
        pltpu.SemaphoreType.DMA,
        pltpu.SemaphoreType.DMA,
    ],
)
def _grid_update(mem_hbm, sk_hbm, sv_hbm, bnd_hbm, out_hbm,
                 buf0, buf1, kbuf0, kbuf1, vbuf0, vbuf1, bvec,
                 sin0, sin1, sout0, sout1, skv0, skv1):
    wid = lax.axis_index("s") * NC + lax.axis_index("c")
    pltpu.sync_copy(bnd_hbm, bvec)
    lanes = lax.iota(jnp.int32, 16)
    bufs = (buf0, buf1)
    kbufs = (kbuf0, kbuf1)
    vbufs = (vbuf0, vbuf1)
    sin = (sin0, sin1)
    sout = (sout0, sout1)
    skv = (skv0, skv1)

    def _bnd(pos):
        return plsc.load_gather(bvec, [jnp.zeros((16,), jnp.int32) + pos])[0]

    def chunk_elems(j):
        return (wid * CPT + j) * CHUNK

    def start_in(j, b):
        pltpu.async_copy(mem_hbm.at[pl.ds(chunk_elems(j), CHUNK)],
                         bufs[b], sin[b])

    def wait_in(j, b):
        pltpu.make_async_copy(mem_hbm.at[pl.ds(chunk_elems(j), CHUNK)],
                              bufs[b], sin[b]).wait()

    def start_out(j, b):
        pltpu.async_copy(bufs[b], out_hbm.at[pl.ds(chunk_elems(j), CHUNK)],
                         sout[b])

    def wait_out(j, b):
        pltpu.make_async_copy(bufs[b],
                              out_hbm.at[pl.ds(chunk_elems(j), CHUNK)],
                              sout[b]).wait()

    def start_kv(base, p):
        pltpu.async_copy(sk_hbm.at[pl.ds(base, KPAD)], kbufs[p], skv[p])
        pltpu.async_copy(sv_hbm.at[pl.ds(base, KPAD)], vbufs[p], skv[p])

    def wait_kv(base, p):
        pltpu.make_async_copy(sk_hbm.at[pl.ds(base, KPAD)],
                              kbufs[p], skv[p]).wait()
        pltpu.make_async_copy(sv_hbm.at[pl.ds(base, KPAD)],
                              vbufs[p], skv[p]).wait()

    def process_chunk(j, b):
        kf = wid * CPT + j
        buf = bufs[b]
        cg0 = kf * (CELLS // 128)

        for t in range(T):
            s = _bnd(t * NQ + kf)
            e = _bnd(t * NQ + kf + 1)
            astart = (s // 16) * 16
            nb = (e - astart + (BATCH - 1)) // BATCH
            tbase = t * G3

            @pl.when(nb > 0)
            def _(t=t, s=s, e=e, astart=astart, nb=nb, tbase=tbase):
                start_kv(astart, 0)

                def win_pair(ibb, _):
                    for p in range(2):
                        ib = ibb + p

                        @pl.when(ib < nb)
                        def _(ib=ib, p=p):
                            base = astart + ib * BATCH
                            wait_kv(base, p)

                            @pl.when(ib + 1 < nb)
                            def _():
                                start_kv(astart + (ib + 1) * BATCH, 1 - p)

                            kbuf = kbufs[p]
                            vbuf = vbufs[p]

                            def vec_body(i, _):
                                off = i * 16
                                ka = kbuf[pl.ds(off, 16)]
                                kb = plsc.load_gather(kbuf, [off + 1 + lanes])
                                v = vbuf[pl.ds(off, 16)]
                                pos = (base + off) + lanes
                                m = (ka != kb) & (pos >= s) & (pos < e)
                                ca = ka - tbase
                                lidx = (((ca >> 7) - cg0) << 10) + (t << 7) \
                                    + (ca & 127)
                                lidx = jnp.minimum(
                                    jnp.maximum(lidx, 0), CHUNK - 1)
                                g = plsc.load_gather(buf, [lidx])
                                w = jnp.maximum(g * jnp.float32(DECAY), v)
                                plsc.store_scatter(buf, [lidx], w, mask=m)
                                return 0

                            lax.fori_loop(0, BATCH // 16, vec_body, 0)
                    return 0

                lax.fori_loop(0, (nb + 1) // 2,
                              lambda q, c: win_pair(q * 2, c), 0)

    start_in(0, 0)

    def ring_pair(jj, _):
        for b in range(2):
            j = jj + b
            wait_in(j, b)

            @pl.when((j + 1 < CPT) & (j >= 1))
            def _():
                wait_out(j - 1, 1 - b)

            @pl.when(j + 1 < CPT)
            def _():
                start_in(j + 1, 1 - b)

            process_chunk(j, b)
            start_out(j, b)
        return 0

    lax.fori_loop(0, CPT // 2, lambda q, c: ring_pair(q * 2, c), 0)
    wait_out(CPT - 2, 0)
    wait_out(CPT - 1, 1)


def kernel(mem, idx, val):
    keys = (idx.astype(jnp.int32)
            + (jnp.arange(T, dtype=jnp.int32) * G3)[:, None]).reshape(-1)
    sk, sv = lax.sort((keys, val.reshape(-1)), is_stable=False, num_keys=1)
    queries = (jnp.arange(T, dtype=jnp.int32)[:, None] * G3
               + jnp.arange(NQ, dtype=jnp.int32)[None, :] * CELLS).reshape(-1)
    bnd = jnp.searchsorted(sk, queries, side="left").astype(jnp.int32)
    bnd = jnp.concatenate([bnd, jnp.zeros((BPAD - T * NQ,), jnp.int32)])
    skp = jnp.concatenate([sk, jnp.full((KPAD,), SENTINEL, jnp.int32)])
    svp = jnp.concatenate([sv, jnp.zeros((KPAD,), jnp.float32)])
    mem_t = mem.reshape(T, G3 // 128, 128).transpose(1, 0, 2).reshape(-1)
    out = _grid_update(mem_t, skp, svp, bnd)
    return out.reshape(G3 // 128, T, 128).transpose(1, 0, 2).reshape(T, G3)


# speedup vs baseline: 1.3407x; 1.0489x over previous
import functools

import jax
import jax.numpy as jnp
from jax import lax
from jax.experimental import pallas as pl
from jax.experimental.pallas import tpu as pltpu
from jax.experimental.pallas import tpu_sc as plsc

T = 8
G3 = 128 ** 3
N = T * G3
U = T * (G3 // 4)
DECAY = 0.95

NC, NS = 2, 16
NW = NC * NS
CHUNK = 32768
NCHUNK = N // CHUNK
CPT = NCHUNK // NW
CELLS = CHUNK // T
BATCH = 1024
KPAD = BATCH + 64
SENTINEL = 0x7FFFFFFF
NQ = NCHUNK + 1
WSZ = 2048
LB = CPT + 1
BSZ = T * LB
BPAD = 160

_MESH = plsc.VectorSubcoreMesh(core_axis_name="c", subcore_axis_name="s")


@functools.partial(
    pl.kernel,
    mesh=_MESH,
    compiler_params=pltpu.CompilerParams(needs_layout_passes=False),
    out_type=jax.ShapeDtypeStruct((N,), jnp.float32),
    scratch_types=[
        pltpu.VMEM((CHUNK,), jnp.float32),
        pltpu.VMEM((CHUNK,), jnp.float32),
        pltpu.VMEM((KPAD,), jnp.int32),
        pltpu.VMEM((KPAD,), jnp.int32),
        pltpu.VMEM((KPAD,), jnp.float32),
        pltpu.VMEM((KPAD,), jnp.float32),
        pltpu.VMEM((BPAD,), jnp.int32),
        pltpu.VMEM((T * WSZ,), jnp.int32),
        pltpu.SemaphoreType.DMA,
        pltpu.SemaphoreType.DMA,
        pltpu.SemaphoreType.DMA,
        pltpu.SemaphoreType.DMA,
        pltpu.SemaphoreType.DMA,
        pltpu.SemaphoreType.DMA,
    ],
)
def _grid_update(mem_hbm, sk_hbm, sv_hbm, out_hbm,
                 buf0, buf1, kbuf0, kbuf1, vbuf0, vbuf1, bvec, sbuf,
                 sin0, sin1, sout0, sout1, skv0, skv1):
    wid = lax.axis_index("s") * NC + lax.axis_index("c")
    lanes = lax.iota(jnp.int32, 16)
    bufs = (buf0, buf1)
    kbufs = (kbuf0, kbuf1)
    vbufs = (vbuf0, vbuf1)
    sin = (sin0, sin1)
    sout = (sout0, sout1)
    skv = (skv0, skv1)

    def _bnd(pos):
        return plsc.load_gather(bvec, [jnp.zeros((16,), jnp.int32) + pos])[0]

    PMAX = U + KPAD - WSZ

    def bound_pass(jj, _):
        kf = wid * CPT + jj
        qv = (lanes % T) * G3 + kf * CELLS
        p0 = jnp.clip(((qv >> 2) - WSZ // 2) & ~7, 0, PMAX)

        def search_iter(carry):
            p0, _, _ = carry
            for t in range(T):
                pltpu.sync_copy(
                    sk_hbm.at[pl.ds(pl.multiple_of(p0[t], 8), WSZ)],
                    sbuf.at[pl.ds(t * WSZ, WSZ)])
            lo = jnp.zeros((16,), jnp.int32)
            hi = jnp.zeros((16,), jnp.int32) + WSZ
            woff = (lanes % T) * WSZ
            for _i in range(11):
                mid = (lo + hi) >> 1
                key = plsc.load_gather(sbuf, [woff + jnp.minimum(mid, WSZ - 1)])
                pred = key < qv
                lo = jnp.where(pred, mid + 1, lo)
                hi = jnp.where(pred, hi, mid)
            need_l = (lo == 0) & (p0 > 0)
            need_r = lo == WSZ
            ok = ~(need_l | need_r)
            nok = plsc.all_reduce_population_count(ok | (lanes >= T))[0]
            p0n = jnp.where(need_l, jnp.maximum(p0 - (WSZ - 8), 0),
                            jnp.where(need_r, jnp.minimum(p0 + (WSZ - 8), PMAX),
                                      p0))
            return p0n, p0 + lo, nok

        def search_cond(carry):
            return carry[2] < 16

        p0f, bound, _ = lax.while_loop(
            search_cond, search_iter,
            (p0, jnp.zeros((16,), jnp.int32), jnp.int32(0)))
        plsc.store_scatter(bvec, [(lanes % T) * LB + jj], bound,
                           mask=lanes < T)
        return 0

    lax.fori_loop(0, LB, bound_pass, 0)

    def chunk_elems(j):
        return (wid * CPT + j) * CHUNK

    def start_in(j, b):
        pltpu.async_copy(mem_hbm.at[pl.ds(chunk_elems(j), CHUNK)],
                         bufs[b], sin[b])

    def wait_in(j, b):
        pltpu.make_async_copy(mem_hbm.at[pl.ds(chunk_elems(j), CHUNK)],
                              bufs[b], sin[b]).wait()

    def start_out(j, b):
        pltpu.async_copy(bufs[b], out_hbm.at[pl.ds(chunk_elems(j), CHUNK)],
                         sout[b])

    def wait_out(j, b):
        pltpu.make_async_copy(bufs[b],
                              out_hbm.at[pl.ds(chunk_elems(j), CHUNK)],
                              sout[b]).wait()

    def start_kv(base, p):
        pltpu.async_copy(sk_hbm.at[pl.ds(base, KPAD)], kbufs[p], skv[p])
        pltpu.async_copy(sv_hbm.at[pl.ds(base, KPAD)], vbufs[p], skv[p])

    def wait_kv(base, p):
        pltpu.make_async_copy(sk_hbm.at[pl.ds(base, KPAD)],
                              kbufs[p], skv[p]).wait()
        pltpu.make_async_copy(sv_hbm.at[pl.ds(base, KPAD)],
                              vbufs[p], skv[p]).wait()

    def process_chunk(j, b):
        kf = wid * CPT + j
        buf = bufs[b]
        cg0 = kf * (CELLS // 128)

        for t in range(T):
            s = _bnd(t * LB + j)
            e = _bnd(t * LB + j + 1)
            astart = (s // 16) * 16
            nb = (e - astart + (BATCH - 1)) // BATCH
            tbase = t * G3

            @pl.when(nb > 0)
            def _(t=t, s=s, e=e, astart=astart, nb=nb, tbase=tbase):
                start_kv(astart, 0)

                def win_pair(ibb, _):
                    for p in range(2):
                        ib = ibb + p

                        @pl.when(ib < nb)
                        def _(ib=ib, p=p):
                            base = astart + ib * BATCH
                            wait_kv(base, p)

                            @pl.when(ib + 1 < nb)
                            def _():
                                start_kv(astart + (ib + 1) * BATCH, 1 - p)

                            kbuf = kbufs[p]
                            vbuf = vbufs[p]

                            def vec_body(i, _):
                                off = i * 16
                                ka = kbuf[pl.ds(off, 16)]
                                kb = plsc.load_gather(kbuf, [off + 1 + lanes])
                                v = vbuf[pl.ds(off, 16)]
                                pos = (base + off) + lanes
                                m = (ka != kb) & (pos >= s) & (pos < e)
                                ca = ka - tbase
                                lidx = (((ca >> 7) - cg0) << 10) + (t << 7) \
                                    + (ca & 127)
                                lidx = jnp.minimum(
                                    jnp.maximum(lidx, 0), CHUNK - 1)
                                g = plsc.load_gather(buf, [lidx])
                                w = jnp.maximum(g * jnp.float32(DECAY), v)
                                plsc.store_scatter(buf, [lidx], w, mask=m)
                                return 0

                            lax.fori_loop(0, BATCH // 16, vec_body, 0)
                    return 0

                lax.fori_loop(0, (nb + 1) // 2,
                              lambda q, c: win_pair(q * 2, c), 0)

    start_in(0, 0)

    def ring_pair(jj, _):
        for b in range(2):
            j = jj + b
            wait_in(j, b)

            @pl.when((j + 1 < CPT) & (j >= 1))
            def _():
                wait_out(j - 1, 1 - b)

            @pl.when(j + 1 < CPT)
            def _():
                start_in(j + 1, 1 - b)

            process_chunk(j, b)
            start_out(j, b)
        return 0

    lax.fori_loop(0, CPT // 2, lambda q, c: ring_pair(q * 2, c), 0)
    wait_out(CPT - 2, 0)
    wait_out(CPT - 1, 1)


def kernel(mem, idx, val):
    keys = (idx.astype(jnp.int32)
            + (jnp.arange(T, dtype=jnp.int32) * G3)[:, None]).reshape(-1)
    sk, sv = lax.sort((keys, val.reshape(-1)), is_stable=False, num_keys=1)
    skp = jnp.concatenate([sk, jnp.full((KPAD,), SENTINEL, jnp.int32)])
    svp = jnp.concatenate([sv, jnp.zeros((KPAD,), jnp.float32)])
    mem_t = mem.reshape(T, G3 // 128, 128).transpose(1, 0, 2).reshape(-1)
    out = _grid_update(mem_t, skp, svp)
    return out.reshape(G3 // 128, T, 128).transpose(1, 0, 2).reshape(T, G3)


# speedup vs baseline: 1.4095x; 1.0513x over previous
import functools

import jax
import jax.numpy as jnp
from jax import lax
from jax.experimental import pallas as pl
from jax.experimental.pallas import tpu as pltpu
from jax.experimental.pallas import tpu_sc as plsc

T = 8
G3 = 128 ** 3
N = T * G3
U = T * (G3 // 4)
DECAY = 0.95

NC, NS = 2, 16
NW = NC * NS
CHUNK = 32768
NCHUNK = N // CHUNK
CPT = NCHUNK // NW
CELLS = CHUNK // T
BATCH = 1024
KPAD = BATCH + 64
SENTINEL = 0x7FFFFFFF
NQ = NCHUNK + 1
WSZ = 2048
LB = CPT + 1
BSZ = T * LB
BPAD = 160

_MESH = plsc.VectorSubcoreMesh(core_axis_name="c", subcore_axis_name="s")


@functools.partial(
    pl.kernel,
    mesh=_MESH,
    compiler_params=pltpu.CompilerParams(needs_layout_passes=False),
    out_type=jax.ShapeDtypeStruct((N,), jnp.float32),
    scratch_types=[
        pltpu.VMEM((CHUNK,), jnp.float32),
        pltpu.VMEM((CHUNK,), jnp.float32),
        pltpu.VMEM((KPAD,), jnp.int32),
        pltpu.VMEM((KPAD,), jnp.int32),
        pltpu.VMEM((KPAD,), jnp.float32),
        pltpu.VMEM((KPAD,), jnp.float32),
        pltpu.VMEM((BPAD,), jnp.int32),
        pltpu.VMEM((T * WSZ,), jnp.int32),
        pltpu.SemaphoreType.DMA,
        pltpu.SemaphoreType.DMA,
        pltpu.SemaphoreType.DMA,
        pltpu.SemaphoreType.DMA,
        pltpu.SemaphoreType.DMA,
        pltpu.SemaphoreType.DMA,
    ],
)
def _grid_update(mem_hbm, sk_hbm, sv_hbm, out_hbm,
                 buf0, buf1, kbuf0, kbuf1, vbuf0, vbuf1, bvec, sbuf,
                 sin0, sin1, sout0, sout1, skv0, skv1):
    wid = lax.axis_index("s") * NC + lax.axis_index("c")
    lanes = lax.iota(jnp.int32, 16)
    bufs = (buf0, buf1)
    kbufs = (kbuf0, kbuf1)
    vbufs = (vbuf0, vbuf1)
    sin = (sin0, sin1)
    sout = (sout0, sout1)
    skv = (skv0, skv1)

    def _bnd(pos):
        return plsc.load_gather(bvec, [jnp.zeros((16,), jnp.int32) + pos])[0]

    PMAX = U + KPAD - WSZ

    def bound_pass(jj, _):
        kf = wid * CPT + jj
        qv = (lanes % T) * G3 + kf * CELLS
        p0 = jnp.clip(((qv >> 2) - WSZ // 2) & ~7, 0, PMAX)

        def search_iter(carry):
            p0, _, _ = carry
            for t in range(T):
                pltpu.async_copy(
                    sk_hbm.at[pl.ds(pl.multiple_of(p0[t], 8), WSZ)],
                    sbuf.at[pl.ds(t * WSZ, WSZ)], skv0)
            for t in range(T):
                pltpu.make_async_copy(
                    sk_hbm.at[pl.ds(pl.multiple_of(p0[t], 8), WSZ)],
                    sbuf.at[pl.ds(t * WSZ, WSZ)], skv0).wait()
            lo = jnp.zeros((16,), jnp.int32)
            hi = jnp.zeros((16,), jnp.int32) + WSZ
            woff = (lanes % T) * WSZ
            for _i in range(11):
                mid = (lo + hi) >> 1
                key = plsc.load_gather(sbuf, [woff + jnp.minimum(mid, WSZ - 1)])
                pred = key < qv
                lo = jnp.where(pred, mid + 1, lo)
                hi = jnp.where(pred, hi, mid)
            need_l = (lo == 0) & (p0 > 0)
            need_r = lo == WSZ
            ok = ~(need_l | need_r)
            nok = plsc.all_reduce_population_count(ok | (lanes >= T))[0]
            p0n = jnp.where(need_l, jnp.maximum(p0 - (WSZ - 8), 0),
                            jnp.where(need_r, jnp.minimum(p0 + (WSZ - 8), PMAX),
                                      p0))
            return p0n, p0 + lo, nok

        def search_cond(carry):
            return carry[2] < 16

        p0f, bound, _ = lax.while_loop(
            search_cond, search_iter,
            (p0, jnp.zeros((16,), jnp.int32), jnp.int32(0)))
        plsc.store_scatter(bvec, [(lanes % T) * LB + jj], bound,
                           mask=lanes < T)
        return 0


    def chunk_elems(j):
        return (wid * CPT + j) * CHUNK

    def start_in(j, b):
        pltpu.async_copy(mem_hbm.at[pl.ds(chunk_elems(j), CHUNK)],
                         bufs[b], sin[b])

    def wait_in(j, b):
        pltpu.make_async_copy(mem_hbm.at[pl.ds(chunk_elems(j), CHUNK)],
                              bufs[b], sin[b]).wait()

    def start_out(j, b):
        pltpu.async_copy(bufs[b], out_hbm.at[pl.ds(chunk_elems(j), CHUNK)],
                         sout[b])

    def wait_out(j, b):
        pltpu.make_async_copy(bufs[b],
                              out_hbm.at[pl.ds(chunk_elems(j), CHUNK)],
                              sout[b]).wait()

    def start_kv(base, p):
        pltpu.async_copy(sk_hbm.at[pl.ds(base, KPAD)], kbufs[p], skv[p])
        pltpu.async_copy(sv_hbm.at[pl.ds(base, KPAD)], vbufs[p], skv[p])

    def wait_kv(base, p):
        pltpu.make_async_copy(sk_hbm.at[pl.ds(base, KPAD)],
                              kbufs[p], skv[p]).wait()
        pltpu.make_async_copy(sv_hbm.at[pl.ds(base, KPAD)],
                              vbufs[p], skv[p]).wait()

    def process_chunk(j, b):
        kf = wid * CPT + j
        buf = bufs[b]
        cg0 = kf * (CELLS // 128)

        for t in range(T):
            s = _bnd(t * LB + j)
            e = _bnd(t * LB + j + 1)
            astart = (s // 16) * 16
            nb = (e - astart + (BATCH - 1)) // BATCH
            tbase = t * G3

            @pl.when(nb > 0)
            def _(t=t, s=s, e=e, astart=astart, nb=nb, tbase=tbase):
                start_kv(astart, 0)

                def win_pair(ibb, _):
                    for p in range(2):
                        ib = ibb + p

                        @pl.when(ib < nb)
                        def _(ib=ib, p=p):
                            base = astart + ib * BATCH
                            wait_kv(base, p)

                            @pl.when(ib + 1 < nb)
                            def _():
                                start_kv(astart + (ib + 1) * BATCH, 1 - p)

                            kbuf = kbufs[p]
                            vbuf = vbufs[p]
                            nvec = (jnp.minimum(e, base + BATCH)
                                    - base + 15) >> 4

                            def vec_body(i, _):
                                off = i * 16
                                ka = kbuf[pl.ds(off, 16)]
                                kb = plsc.load_gather(kbuf, [off + 1 + lanes])
                                v = vbuf[pl.ds(off, 16)]
                                pos = (base + off) + lanes
                                m = (ka != kb) & (pos >= s) & (pos < e)
                                ca = ka - tbase
                                lidx = (((ca >> 7) - cg0) << 10) + (t << 7) \
                                    + (ca & 127)
                                lidx = jnp.minimum(
                                    jnp.maximum(lidx, 0), CHUNK - 1)
                                g = plsc.load_gather(buf, [lidx])
                                w = jnp.maximum(g * jnp.float32(DECAY), v)
                                plsc.store_scatter(buf, [lidx], w, mask=m)
                                return 0

                            lax.fori_loop(0, nvec, vec_body, 0)
                    return 0

                lax.fori_loop(0, (nb + 1) // 2,
                              lambda q, c: win_pair(q * 2, c), 0)

    start_in(0, 0)
    lax.fori_loop(0, LB, bound_pass, 0)

    def ring_pair(jj, _):
        for b in range(2):
            j = jj + b
            wait_in(j, b)

            @pl.when((j + 1 < CPT) & (j >= 1))
            def _():
                wait_out(j - 1, 1 - b)

            @pl.when(j + 1 < CPT)
            def _():
                start_in(j + 1, 1 - b)

            process_chunk(j, b)
            start_out(j, b)
        return 0

    lax.fori_loop(0, CPT // 2, lambda q, c: ring_pair(q * 2, c), 0)
    wait_out(CPT - 2, 0)
    wait_out(CPT - 1, 1)


def kernel(mem, idx, val):
    keys = (idx.astype(jnp.int32)
            + (jnp.arange(T, dtype=jnp.int32) * G3)[:, None]).reshape(-1)
    sk, sv = lax.sort((keys, val.reshape(-1)), is_stable=False, num_keys=1)
    skp = jnp.concatenate([sk, jnp.full((KPAD,), SENTINEL, jnp.int32)])
    svp = jnp.concatenate([sv, jnp.zeros((KPAD,), jnp.float32)])
    mem_t = mem.reshape(T, G3 // 128, 128).transpose(1, 0, 2).reshape(-1)
    out = _grid_update(mem_t, skp, svp)
    return out.reshape(G3 // 128, T, 128).transpose(1, 0, 2).reshape(T, G3)


# speedup vs baseline: 1.4231x; 1.0097x over previous
import functools

import jax
import jax.numpy as jnp
from jax import lax
from jax.experimental import pallas as pl
from jax.experimental.pallas import tpu as pltpu
from jax.experimental.pallas import tpu_sc as plsc

T = 8
G3 = 128 ** 3
N = T * G3
U = T * (G3 // 4)
DECAY = 0.95

NC, NS = 2, 16
NW = NC * NS
CHUNK = 32768
NCHUNK = N // CHUNK
CPT = NCHUNK // NW
CELLS = CHUNK // T
BATCH = 1024
KPAD = BATCH + 64
SENTINEL = 0x7FFFFFFF
NQ = NCHUNK + 1
WSZ = 2048
LB = CPT + 1
BSZ = T * LB
BPAD = 160

_MESH = plsc.VectorSubcoreMesh(core_axis_name="c", subcore_axis_name="s")


@functools.partial(
    pl.kernel,
    mesh=_MESH,
    compiler_params=pltpu.CompilerParams(needs_layout_passes=False),
    out_type=jax.ShapeDtypeStruct((N,), jnp.float32),
    scratch_types=[
        pltpu.VMEM((CHUNK,), jnp.float32),
        pltpu.VMEM((CHUNK,), jnp.float32),
        pltpu.VMEM((KPAD,), jnp.int32),
        pltpu.VMEM((KPAD,), jnp.int32),
        pltpu.VMEM((KPAD,), jnp.float32),
        pltpu.VMEM((KPAD,), jnp.float32),
        pltpu.VMEM((BPAD,), jnp.int32),
        pltpu.VMEM((T * WSZ,), jnp.int32),
        pltpu.SemaphoreType.DMA,
        pltpu.SemaphoreType.DMA,
        pltpu.SemaphoreType.DMA,
        pltpu.SemaphoreType.DMA,
        pltpu.SemaphoreType.DMA,
        pltpu.SemaphoreType.DMA,
    ],
)
def _grid_update(mem_hbm, sk_hbm, sv_hbm, out_hbm,
                 buf0, buf1, kbuf0, kbuf1, vbuf0, vbuf1, bvec, sbuf,
                 sin0, sin1, sout0, sout1, skv0, skv1):
    wid = lax.axis_index("s") * NC + lax.axis_index("c")
    lanes = lax.iota(jnp.int32, 16)
    bufs = (buf0, buf1)
    kbufs = (kbuf0, kbuf1)
    vbufs = (vbuf0, vbuf1)
    sin = (sin0, sin1)
    sout = (sout0, sout1)
    skv = (skv0, skv1)

    def _bnd(pos):
        return plsc.load_gather(bvec, [jnp.zeros((16,), jnp.int32) + pos])[0]

    PMAX = U + KPAD - WSZ

    def bound_pass(jj, _):
        kf = wid * CPT + jj
        qv = (lanes % T) * G3 + kf * CELLS
        p0 = jnp.clip(((qv >> 2) - WSZ // 2) & ~7, 0, PMAX)

        def search_iter(carry):
            p0, _, _ = carry
            for t in range(T):
                pltpu.async_copy(
                    sk_hbm.at[pl.ds(pl.multiple_of(p0[t], 8), WSZ)],
                    sbuf.at[pl.ds(t * WSZ, WSZ)], skv0)
            for t in range(T):
                pltpu.make_async_copy(
                    sk_hbm.at[pl.ds(pl.multiple_of(p0[t], 8), WSZ)],
                    sbuf.at[pl.ds(t * WSZ, WSZ)], skv0).wait()
            lo = jnp.zeros((16,), jnp.int32)
            hi = jnp.zeros((16,), jnp.int32) + WSZ
            woff = (lanes % T) * WSZ
            for _i in range(11):
                mid = (lo + hi) >> 1
                key = plsc.load_gather(sbuf, [woff + jnp.minimum(mid, WSZ - 1)])
                pred = key < qv
                lo = jnp.where(pred, mid + 1, lo)
                hi = jnp.where(pred, hi, mid)
            need_l = (lo == 0) & (p0 > 0)
            need_r = lo == WSZ
            ok = ~(need_l | need_r)
            nok = plsc.all_reduce_population_count(ok | (lanes >= T))[0]
            p0n = jnp.where(need_l, jnp.maximum(p0 - (WSZ - 8), 0),
                            jnp.where(need_r, jnp.minimum(p0 + (WSZ - 8), PMAX),
                                      p0))
            return p0n, p0 + lo, nok

        def search_cond(carry):
            return carry[2] < 16

        p0f, bound, _ = lax.while_loop(
            search_cond, search_iter,
            (p0, jnp.zeros((16,), jnp.int32), jnp.int32(0)))
        plsc.store_scatter(bvec, [(lanes % T) * LB + jj], bound,
                           mask=lanes < T)
        return 0


    def chunk_elems(j):
        return (wid * CPT + j) * CHUNK

    def start_in(j, b):
        pltpu.async_copy(mem_hbm.at[pl.ds(chunk_elems(j), CHUNK)],
                         bufs[b], sin[b])

    def wait_in(j, b):
        pltpu.make_async_copy(mem_hbm.at[pl.ds(chunk_elems(j), CHUNK)],
                              bufs[b], sin[b]).wait()

    def start_out(j, b):
        pltpu.async_copy(bufs[b], out_hbm.at[pl.ds(chunk_elems(j), CHUNK)],
                         sout[b])

    def wait_out(j, b):
        pltpu.make_async_copy(bufs[b],
                              out_hbm.at[pl.ds(chunk_elems(j), CHUNK)],
                              sout[b]).wait()

    def start_kv(base, p):
        pltpu.async_copy(sk_hbm.at[pl.ds(base, KPAD)], kbufs[p], skv[p])
        pltpu.async_copy(sv_hbm.at[pl.ds(base, KPAD)], vbufs[p], skv[p])

    def wait_kv(base, p):
        pltpu.make_async_copy(sk_hbm.at[pl.ds(base, KPAD)],
                              kbufs[p], skv[p]).wait()
        pltpu.make_async_copy(sv_hbm.at[pl.ds(base, KPAD)],
                              vbufs[p], skv[p]).wait()

    def process_chunk(j, b):
        kf = wid * CPT + j
        buf = bufs[b]
        cg0 = kf * (CELLS // 128)

        for t in range(T):
            p = t % 2
            s = _bnd(t * LB + j)
            e = _bnd(t * LB + j + 1)
            astart = (s // 16) * 16
            tbase = t * G3
            wait_kv(astart, p)

            if t < T - 1:
                sn = _bnd((t + 1) * LB + j)
                start_kv((sn // 16) * 16, 1 - p)
            else:
                @pl.when(j + 1 < CPT)
                def _():
                    sn = _bnd(j + 1)
                    start_kv((sn // 16) * 16, 1 - p)

            def proc_window(base, pp, s=s, e=e, t=t, tbase=tbase, cg0=cg0,
                            buf=buf):
                kbuf = kbufs[pp]
                vbuf = vbufs[pp]
                nvec = (jnp.minimum(e, base + BATCH) - base + 15) >> 4

                def vec_body(i, _):
                    off = i * 16
                    ka = kbuf[pl.ds(off, 16)]
                    kb = plsc.load_gather(kbuf, [off + 1 + lanes])
                    v = vbuf[pl.ds(off, 16)]
                    pos = (base + off) + lanes
                    m = (ka != kb) & (pos >= s) & (pos < e)
                    ca = ka - tbase
                    lidx = (((ca >> 7) - cg0) << 10) + (t << 7) + (ca & 127)
                    lidx = jnp.minimum(jnp.maximum(lidx, 0), CHUNK - 1)
                    g = plsc.load_gather(buf, [lidx])
                    w = jnp.maximum(g * jnp.float32(DECAY), v)
                    plsc.store_scatter(buf, [lidx], w, mask=m)
                    return 0

                lax.fori_loop(0, nvec, vec_body, 0)

            proc_window(astart, p)

            def extra_cond(ib):
                return astart + ib * BATCH < e

            def extra_body(ib, p=p, astart=astart):
                base = astart + ib * BATCH
                start_kv(base, p)
                wait_kv(base, p)
                proc_window(base, p)
                return ib + 1

            lax.while_loop(extra_cond, extra_body, jnp.int32(1))

    start_in(0, 0)
    lax.fori_loop(0, LB, bound_pass, 0)
    start_kv((_bnd(0) // 16) * 16, 0)

    def ring_pair(jj, _):
        for b in range(2):
            j = jj + b
            wait_in(j, b)

            @pl.when((j + 1 < CPT) & (j >= 1))
            def _():
                wait_out(j - 1, 1 - b)

            @pl.when(j + 1 < CPT)
            def _():
                start_in(j + 1, 1 - b)

            process_chunk(j, b)
            start_out(j, b)
        return 0

    lax.fori_loop(0, CPT // 2, lambda q, c: ring_pair(q * 2, c), 0)
    wait_out(CPT - 2, 0)
    wait_out(CPT - 1, 1)


def kernel(mem, idx, val):
    keys = (idx.astype(jnp.int32)
            + (jnp.arange(T, dtype=jnp.int32) * G3)[:, None]).reshape(-1)
    sk, sv = lax.sort((keys, val.reshape(-1)), is_stable=False, num_keys=1)
    skp = jnp.concatenate([sk, jnp.full((KPAD,), SENTINEL, jnp.int32)])
    svp = jnp.concatenate([sv, jnp.zeros((KPAD,), jnp.float32)])
    mem_t = mem.reshape(T, G3 // 128, 128).transpose(1, 0, 2).reshape(-1)
    out = _grid_update(mem_t, skp, svp)
    return out.reshape(G3 // 128, T, 128).transpose(1, 0, 2).reshape(T, G3)


# speedup vs baseline: 1.4433x; 1.0142x over previous
import functools

import jax
import jax.numpy as jnp
from jax import lax
from jax.experimental import pallas as pl
from jax.experimental.pallas import tpu as pltpu
from jax.experimental.pallas import tpu_sc as plsc

T = 8
G3 = 128 ** 3
N = T * G3
U = T * (G3 // 4)
DECAY = 0.95

NC, NS = 2, 16
NW = NC * NS
CHUNK = 32768
NCHUNK = N // CHUNK
CPT = NCHUNK // NW
CELLS = CHUNK // T
BATCH = 1152
KPAD = BATCH + 64
SENTINEL = 0x7FFFFFFF
NQ = NCHUNK + 1
WSZ = 2048
LB = CPT + 1
BSZ = T * LB
BPAD = 160

_MESH = plsc.VectorSubcoreMesh(core_axis_name="c", subcore_axis_name="s")


@functools.partial(
    pl.kernel,
    mesh=_MESH,
    compiler_params=pltpu.CompilerParams(needs_layout_passes=False),
    out_type=jax.ShapeDtypeStruct((N,), jnp.float32),
    scratch_types=[
        pltpu.VMEM((CHUNK,), jnp.float32),
        pltpu.VMEM((CHUNK,), jnp.float32),
        pltpu.VMEM((KPAD,), jnp.int32),
        pltpu.VMEM((KPAD,), jnp.int32),
        pltpu.VMEM((KPAD,), jnp.float32),
        pltpu.VMEM((KPAD,), jnp.float32),
        pltpu.VMEM((BPAD,), jnp.int32),
        pltpu.VMEM((T * WSZ,), jnp.int32),
        pltpu.SemaphoreType.DMA,
        pltpu.SemaphoreType.DMA,
        pltpu.SemaphoreType.DMA,
        pltpu.SemaphoreType.DMA,
        pltpu.SemaphoreType.DMA,
        pltpu.SemaphoreType.DMA,
    ],
)
def _grid_update(mem_hbm, sk_hbm, sv_hbm, out_hbm,
                 buf0, buf1, kbuf0, kbuf1, vbuf0, vbuf1, bvec, sbuf,
                 sin0, sin1, sout0, sout1, skv0, skv1):
    wid = lax.axis_index("s") * NC + lax.axis_index("c")
    lanes = lax.iota(jnp.int32, 16)
    bufs = (buf0, buf1)
    kbufs = (kbuf0, kbuf1)
    vbufs = (vbuf0, vbuf1)
    sin = (sin0, sin1)
    sout = (sout0, sout1)
    skv = (skv0, skv1)

    def _bnd(pos):
        return plsc.load_gather(bvec, [jnp.zeros((16,), jnp.int32) + pos])[0]

    PMAX = U + KPAD - WSZ

    def bound_pass(jj, _):
        kf = wid * CPT + jj
        qv = (lanes % T) * G3 + kf * CELLS
        p0 = jnp.clip(((qv >> 2) - WSZ // 2) & ~7, 0, PMAX)

        def search_iter(carry):
            p0, _, _ = carry
            for t in range(T):
                pltpu.async_copy(
                    sk_hbm.at[pl.ds(pl.multiple_of(p0[t], 8), WSZ)],
                    sbuf.at[pl.ds(t * WSZ, WSZ)], skv0)
            for t in range(T):
                pltpu.make_async_copy(
                    sk_hbm.at[pl.ds(pl.multiple_of(p0[t], 8), WSZ)],
                    sbuf.at[pl.ds(t * WSZ, WSZ)], skv0).wait()
            lo = jnp.zeros((16,), jnp.int32)
            hi = jnp.zeros((16,), jnp.int32) + WSZ
            woff = (lanes % T) * WSZ
            for _i in range(11):
                mid = (lo + hi) >> 1
                key = plsc.load_gather(sbuf, [woff + jnp.minimum(mid, WSZ - 1)])
                pred = key < qv
                lo = jnp.where(pred, mid + 1, lo)
                hi = jnp.where(pred, hi, mid)
            need_l = (lo == 0) & (p0 > 0)
            need_r = lo == WSZ
            ok = ~(need_l | need_r)
            nok = plsc.all_reduce_population_count(ok | (lanes >= T))[0]
            p0n = jnp.where(need_l, jnp.maximum(p0 - (WSZ - 8), 0),
                            jnp.where(need_r, jnp.minimum(p0 + (WSZ - 8), PMAX),
                                      p0))
            return p0n, p0 + lo, nok

        def search_cond(carry):
            return carry[2] < 16

        p0f, bound, _ = lax.while_loop(
            search_cond, search_iter,
            (p0, jnp.zeros((16,), jnp.int32), jnp.int32(0)))
        plsc.store_scatter(bvec, [(lanes % T) * LB + jj], bound,
                           mask=lanes < T)
        return 0


    def chunk_elems(j):
        return (wid * CPT + j) * CHUNK

    def start_in(j, b):
        pltpu.async_copy(mem_hbm.at[pl.ds(chunk_elems(j), CHUNK)],
                         bufs[b], sin[b])

    def wait_in(j, b):
        pltpu.make_async_copy(mem_hbm.at[pl.ds(chunk_elems(j), CHUNK)],
                              bufs[b], sin[b]).wait()

    def start_out(j, b):
        pltpu.async_copy(bufs[b], out_hbm.at[pl.ds(chunk_elems(j), CHUNK)],
                         sout[b])

    def wait_out(j, b):
        pltpu.make_async_copy(bufs[b],
                              out_hbm.at[pl.ds(chunk_elems(j), CHUNK)],
                              sout[b]).wait()

    def start_kv(base, p):
        pltpu.async_copy(sk_hbm.at[pl.ds(base, KPAD)], kbufs[p], skv[p])
        pltpu.async_copy(sv_hbm.at[pl.ds(base, KPAD)], vbufs[p], skv[p])

    def wait_kv(base, p):
        pltpu.make_async_copy(sk_hbm.at[pl.ds(base, KPAD)],
                              kbufs[p], skv[p]).wait()
        pltpu.make_async_copy(sv_hbm.at[pl.ds(base, KPAD)],
                              vbufs[p], skv[p]).wait()

    def process_chunk(j, b):
        kf = wid * CPT + j
        buf = bufs[b]
        cg0 = kf * (CELLS // 128)

        for t in range(T):
            p = t % 2
            s = _bnd(t * LB + j)
            e = _bnd(t * LB + j + 1)
            astart = (s // 16) * 16
            tbase = t * G3
            wait_kv(astart, p)

            if t < T - 1:
                sn = _bnd((t + 1) * LB + j)
                start_kv((sn // 16) * 16, 1 - p)
            else:
                @pl.when(j + 1 < CPT)
                def _():
                    sn = _bnd(j + 1)
                    start_kv((sn // 16) * 16, 1 - p)

            def proc_window(base, pp, s=s, e=e, t=t, tbase=tbase, cg0=cg0,
                            buf=buf):
                kbuf = kbufs[pp]
                vbuf = vbufs[pp]
                nvec = (jnp.minimum(e, base + BATCH) - base + 15) >> 4

                def vec_body(i, _):
                    off = i * 16
                    ka = kbuf[pl.ds(off, 16)]
                    kb = plsc.load_gather(kbuf, [off + 1 + lanes])
                    v = vbuf[pl.ds(off, 16)]
                    pos = (base + off) + lanes
                    m = (ka != kb) & (pos >= s) & (pos < e)
                    ca = ka - tbase
                    lidx = (((ca >> 7) - cg0) << 10) + (t << 7) + (ca & 127)
                    lidx = jnp.minimum(jnp.maximum(lidx, 0), CHUNK - 1)
                    g = plsc.load_gather(buf, [lidx])
                    w = jnp.maximum(g * jnp.float32(DECAY), v)
                    plsc.store_scatter(buf, [lidx], w, mask=m)
                    return 0

                lax.fori_loop(0, nvec, vec_body, 0)

            proc_window(astart, p)

            def extra_cond(ib):
                return astart + ib * BATCH < e

            def extra_body(ib, p=p, astart=astart):
                base = astart + ib * BATCH
                start_kv(base, p)
                wait_kv(base, p)
                proc_window(base, p)
                return ib + 1

            lax.while_loop(extra_cond, extra_body, jnp.int32(1))

    start_in(0, 0)
    lax.fori_loop(0, LB, bound_pass, 0)
    start_kv((_bnd(0) // 16) * 16, 0)

    def ring_pair(jj, _):
        for b in range(2):
            j = jj + b
            wait_in(j, b)

            @pl.when((j + 1 < CPT) & (j >= 1))
            def _():
                wait_out(j - 1, 1 - b)

            @pl.when(j + 1 < CPT)
            def _():
                start_in(j + 1, 1 - b)

            process_chunk(j, b)
            start_out(j, b)
        return 0

    lax.fori_loop(0, CPT // 2, lambda q, c: ring_pair(q * 2, c), 0)
    wait_out(CPT - 2, 0)
    wait_out(CPT - 1, 1)


def kernel(mem, idx, val):
    keys = (idx.astype(jnp.int32)
            + (jnp.arange(T, dtype=jnp.int32) * G3)[:, None]).reshape(-1)
    sk, sv = lax.sort((keys, val.reshape(-1)), is_stable=False, num_keys=1)
    skp = jnp.concatenate([sk, jnp.full((KPAD,), SENTINEL, jnp.int32)])
    svp = jnp.concatenate([sv, jnp.zeros((KPAD,), jnp.float32)])
    mem_t = mem.reshape(T, G3 // 128, 128).transpose(1, 0, 2).reshape(-1)
    out = _grid_update(mem_t, skp, svp)
    return out.reshape(G3 // 128, T, 128).transpose(1, 0, 2).reshape(T, G3)
